# baseline (speedup 1.0000x reference)
import functools
import math

import jax
import jax.numpy as jnp
from jax import lax
from jax.experimental import pallas as pl
from jax.experimental.pallas import tpu as pltpu
from jax.experimental.pallas import tpu_sc as plsc

EMBED_W = 64
SCALE = math.sqrt(64.0)
LANES = 128
NBUF = 2


def _make_lookup(n_seq: int, n_batch: int, n_vocab: int):
    info = plsc.get_sparse_core_info()
    nc, ns = info.num_cores, info.num_subcores
    nw = nc * ns
    assert n_batch == nw * LANES
    n_outer = n_seq // NBUF
    assert n_outer * NBUF == n_seq

    mesh = plsc.VectorSubcoreMesh(core_axis_name="c", subcore_axis_name="s")

    @functools.partial(
        pl.kernel,
        mesh=mesh,
        out_type=jax.ShapeDtypeStruct(
            (n_seq, EMBED_W // 8, nw, 8, LANES), jnp.float32
        ),
        scratch_types=[
            pltpu.VMEM((n_seq, LANES), jnp.int32),
            pltpu.VMEM((n_seq, LANES), jnp.int32),
            pltpu.VMEM((NBUF, LANES, LANES), jnp.float32),
            pltpu.VMEM((NBUF, 1, EMBED_W // 8, 1, 8, LANES), jnp.float32),
            pltpu.SemaphoreType.DMA,
            pltpu.SemaphoreType.DMA,
            pltpu.SemaphoreType.DMA,
            pltpu.SemaphoreType.DMA,
        ],
        compiler_params=pltpu.CompilerParams(
            use_tc_tiling_on_sc=False, needs_layout_passes=False
        ),
    )
    def lookup(lut2_hbm, xt_hbm, out_hbm, idx_v, pb_v, rows_v, tr_v,
               g0, g1, w0, w1):
        wid = lax.axis_index("s") * nc + lax.axis_index("c")
        g_sems = [g0, g1]
        w_sems = [w0, w1]

        pltpu.sync_copy(xt_hbm.at[:, pl.ds(wid * LANES, LANES)], idx_v)

        def prep(s, carry):
            for k in range(LANES // 16):
                sl = pl.ds(k * 16, 16)
                v = idx_v[s, sl]
                pb_v[s, sl] = (v & 1) << 6
                idx_v[s, sl] = v >> 1
            return carry

        lax.fori_loop(0, n_seq, prep, 0)

        iota16 = lax.iota(jnp.int32, 16)
        rows_idx = [iota16 + k * 16 for k in range(8)]

        for b in range(NBUF):
            pltpu.async_copy(
                lut2_hbm.at[idx_v.at[b]], rows_v.at[b], g_sems[b]
            )

        def outer(t, carry):
            for b in range(NBUF):
                s = t * NBUF + b
                pltpu.make_async_copy(
                    lut2_hbm.at[pl.ds(0, LANES)], rows_v.at[b], g_sems[b]
                ).wait()

                @pl.when(t > 0)
                def _drain(b=b):
                    pltpu.make_async_copy(
                        tr_v.at[b],
                        out_hbm.at[pl.ds(0, 1), :, pl.ds(0, 1)],
                        w_sems[b],
                    ).wait()

                rows_b = rows_v.at[b]
                pbs = [pb_v[s, pl.ds(k * 16, 16)] for k in range(8)]

                def erow(e, inner, b=b, rows_b=rows_b, pbs=pbs):
                    e_hi = e >> 3
                    e_lo = e & 7
                    for k in range(8):
                        val = plsc.load_gather(
                            rows_b, [rows_idx[k], pbs[k] + e]
                        )
                        tr_v[b, 0, e_hi, 0, e_lo, pl.ds(k * 16, 16)] = (
                            val * SCALE
                        )
                    return inner

                lax.fori_loop(0, EMBED_W, erow, 0)

                @pl.when(t + 1 < n_outer)
                def _refill(b=b, s=s):
                    pltpu.async_copy(
                        lut2_hbm.at[idx_v.at[s + NBUF]],
                        rows_v.at[b],
                        g_sems[b],
                    )

                pltpu.async_copy(
                    tr_v.at[b],
                    out_hbm.at[pl.ds(s, 1), :, pl.ds(wid, 1)],
                    w_sems[b],
                )
            return carry

        lax.fori_loop(0, n_outer, outer, 0)

        for b in range(NBUF):
            pltpu.make_async_copy(
                tr_v.at[b],
                out_hbm.at[pl.ds(0, 1), :, pl.ds(0, 1)],
                w_sems[b],
            ).wait()

    return lookup


def kernel(x, lut):
    n_batch, n_seq = x.shape
    n_vocab, embed = lut.shape
    xt = x.T.astype(jnp.int32)
    lut2 = lut.reshape(n_vocab // 2, 2 * embed)
    out5 = _make_lookup(n_seq, n_batch, n_vocab)(lut2, xt)
    return out5.transpose(2, 4, 0, 1, 3).reshape(n_batch, n_seq, embed)


# speedup vs baseline: 1.5243x; 1.5243x over previous
import functools
import math

import jax
import jax.numpy as jnp
from jax import lax
from jax.experimental import pallas as pl
from jax.experimental.pallas import tpu as pltpu
from jax.experimental.pallas import tpu_sc as plsc

EMBED_W = 64
SCALE = math.sqrt(64.0)
LANES = 128
NBUF = 2


def _make_lookup(n_seq: int, n_batch: int, n_vocab: int):
    info = plsc.get_sparse_core_info()
    nc, ns = info.num_cores, info.num_subcores
    nw = nc * ns
    assert n_batch == nw * LANES
    n_outer = n_seq // NBUF
    assert n_outer * NBUF == n_seq

    mesh = plsc.VectorSubcoreMesh(core_axis_name="c", subcore_axis_name="s")

    @functools.partial(
        pl.kernel,
        mesh=mesh,
        out_type=jax.ShapeDtypeStruct((n_batch, n_seq, 2 * EMBED_W), jnp.float32),
        scratch_types=[
            pltpu.VMEM((n_seq, LANES), jnp.int32),
            pltpu.VMEM((n_seq, LANES), jnp.int32),
            pltpu.VMEM((NBUF, LANES, 2 * EMBED_W), jnp.float32),
            pltpu.VMEM((NBUF, LANES, 1, EMBED_W), jnp.float32),
            pltpu.SemaphoreType.DMA,
            pltpu.SemaphoreType.DMA,
            pltpu.SemaphoreType.DMA,
            pltpu.SemaphoreType.DMA,
        ],
        compiler_params=pltpu.CompilerParams(
            use_tc_tiling_on_sc=False, needs_layout_passes=False
        ),
    )
    def lookup(lut2_hbm, idxh_hbm, pb_hbm, out_hbm, idx_v, pb_v,
               rows_v, wb_v, g0, g1, w0, w1):
        wid = lax.axis_index("s") * nc + lax.axis_index("c")
        g_sems = [g0, g1]
        w_sems = [w0, w1]
        b0 = wid * LANES

        pltpu.sync_copy(idxh_hbm.at[:, pl.ds(b0, LANES)], idx_v)
        pltpu.sync_copy(pb_hbm.at[:, pl.ds(b0, LANES)], pb_v)

        for b in range(NBUF):
            pltpu.async_copy(
                lut2_hbm.at[idx_v.at[b]], rows_v.at[b], g_sems[b]
            )

        def outer(t, carry):
            for b in range(NBUF):
                s = t * NBUF + b
                pltpu.make_async_copy(
                    lut2_hbm.at[pl.ds(0, LANES)], rows_v.at[b], g_sems[b]
                ).wait()
                @pl.when(t > 0)
                def _drain(b=b):
                    pltpu.make_async_copy(
                        wb_v.at[b],
                        out_hbm.at[pl.ds(0, LANES), pl.ds(0, 1), pl.ds(0, EMBED_W)],
                        w_sems[b],
                    ).wait()

                def select_grp(g, inner, b=b, s=s):
                    pvec = pb_v[s, pl.ds(g * 16, 16)]
                    for l in range(16):
                        p = pvec[l]
                        j = g * 16 + l
                        for q in range(EMBED_W // 16):
                            src = rows_v[b, j, pl.ds(p + q * 16, 16)]
                            wb_v[b, j, 0, pl.ds(q * 16, 16)] = src * SCALE
                    return inner

                lax.fori_loop(0, LANES // 16, select_grp, 0)

                @pl.when(t + 1 < n_outer)
                def _refill(b=b, s=s):
                    pltpu.async_copy(
                        lut2_hbm.at[idx_v.at[s + NBUF]],
                        rows_v.at[b],
                        g_sems[b],
                    )

                pltpu.async_copy(
                    wb_v.at[b],
                    out_hbm.at[pl.ds(b0, LANES), pl.ds(s, 1), pl.ds(0, EMBED_W)],
                    w_sems[b],
                )
            return carry

        lax.fori_loop(0, n_outer, outer, 0)

        for b in range(NBUF):
            pltpu.make_async_copy(
                wb_v.at[b],
                out_hbm.at[pl.ds(0, LANES), pl.ds(0, 1), pl.ds(0, EMBED_W)],
                w_sems[b],
            ).wait()

    return lookup


def kernel(x, lut):
    n_batch, n_seq = x.shape
    n_vocab, embed = lut.shape
    xi = x.astype(jnp.int32)
    idxh = (xi >> 1).T
    pb = ((xi & 1) << 6).T
    lut2 = lut.reshape(n_vocab // 2, 2 * embed)
    outp = _make_lookup(n_seq, n_batch, n_vocab)(lut2, idxh, pb)
    return outp[:, :, :embed]


# speedup vs baseline: 1.6202x; 1.0629x over previous
import functools
import math

import jax
import jax.numpy as jnp
from jax import lax
from jax.experimental import pallas as pl
from jax.experimental.pallas import tpu as pltpu
from jax.experimental.pallas import tpu_sc as plsc

EMBED_W = 64
SCALE = math.sqrt(64.0)
LANES = 128
NBUF = 2


def _make_lookup(n_seq: int, n_batch: int, n_vocab: int):
    info = plsc.get_sparse_core_info()
    nc, ns = info.num_cores, info.num_subcores
    nw = nc * ns
    assert n_batch == nw * LANES
    n_outer = n_seq // NBUF
    assert n_outer * NBUF == n_seq

    mesh = plsc.VectorSubcoreMesh(core_axis_name="c", subcore_axis_name="s")

    @functools.partial(
        pl.kernel,
        mesh=mesh,
        out_type=jax.ShapeDtypeStruct((n_batch, n_seq, 2 * EMBED_W), jnp.float32),
        scratch_types=[
            pltpu.VMEM((n_seq, LANES), jnp.int32),
            pltpu.VMEM((NBUF, LANES, 2 * EMBED_W), jnp.float32),
            pltpu.VMEM((NBUF, LANES, 1, EMBED_W), jnp.float32),
            pltpu.SemaphoreType.DMA,
            pltpu.SemaphoreType.DMA,
            pltpu.SemaphoreType.DMA,
            pltpu.SemaphoreType.DMA,
        ],
        compiler_params=pltpu.CompilerParams(
            use_tc_tiling_on_sc=False, needs_layout_passes=False
        ),
    )
    def lookup(lutp_hbm, xt_hbm, out_hbm, idx_v, rows_v, wb_v,
               g0, g1, w0, w1):
        wid = lax.axis_index("s") * nc + lax.axis_index("c")
        g_sems = [g0, g1]
        w_sems = [w0, w1]
        b0 = wid * LANES

        pltpu.sync_copy(xt_hbm.at[:, pl.ds(b0, LANES)], idx_v)

        for b in range(NBUF):
            pltpu.async_copy(
                lutp_hbm.at[idx_v.at[b]], rows_v.at[b], g_sems[b]
            )

        def outer(t, carry):
            for b in range(NBUF):
                s = t * NBUF + b
                pltpu.make_async_copy(
                    lutp_hbm.at[pl.ds(0, LANES)], rows_v.at[b], g_sems[b]
                ).wait()

                @pl.when(t > 0)
                def _drain(b=b):
                    pltpu.make_async_copy(
                        wb_v.at[b],
                        out_hbm.at[pl.ds(0, LANES), pl.ds(0, 1),
                                   pl.ds(0, EMBED_W)],
                        w_sems[b],
                    ).wait()

                def scale_row(j, inner, b=b):
                    for q in range(EMBED_W // 16):
                        sl = pl.ds(q * 16, 16)
                        wb_v[b, j, 0, sl] = rows_v[b, j, sl] * SCALE
                    return inner

                lax.fori_loop(0, LANES, scale_row, 0)

                @pl.when(t + 1 < n_outer)
                def _refill(b=b, s=s):
                    pltpu.async_copy(
                        lutp_hbm.at[idx_v.at[s + NBUF]],
                        rows_v.at[b],
                        g_sems[b],
                    )

                pltpu.async_copy(
                    wb_v.at[b],
                    out_hbm.at[pl.ds(b0, LANES), pl.ds(s, 1),
                               pl.ds(0, EMBED_W)],
                    w_sems[b],
                )
            return carry

        lax.fori_loop(0, n_outer, outer, 0)

        for b in range(NBUF):
            pltpu.make_async_copy(
                wb_v.at[b],
                out_hbm.at[pl.ds(0, LANES), pl.ds(0, 1), pl.ds(0, EMBED_W)],
                w_sems[b],
            ).wait()

    return lookup


def kernel(x, lut):
    n_batch, n_seq = x.shape
    n_vocab, embed = lut.shape
    xt = x.T.astype(jnp.int32)
    lutp = jnp.pad(lut, ((0, 0), (0, embed)))
    outp = _make_lookup(n_seq, n_batch, n_vocab)(lutp, xt)
    return outp[:, :, :embed]


# speedup vs baseline: 1.6388x; 1.0114x over previous
import functools
import math

import jax
import jax.numpy as jnp
from jax import lax
from jax.experimental import pallas as pl
from jax.experimental.pallas import tpu as pltpu
from jax.experimental.pallas import tpu_sc as plsc

EMBED_W = 64
SCALE = math.sqrt(64.0)
LANES = 128
NBUF = 2


def _make_lookup(n_seq: int, n_batch: int, n_vocab: int):
    info = plsc.get_sparse_core_info()
    nc, ns = info.num_cores, info.num_subcores
    nw = nc * ns
    assert n_batch == nw * LANES
    n_outer = n_seq // NBUF
    assert n_outer * NBUF == n_seq

    mesh = plsc.VectorSubcoreMesh(core_axis_name="c", subcore_axis_name="s")

    @functools.partial(
        pl.kernel,
        mesh=mesh,
        out_type=jax.ShapeDtypeStruct((n_batch, n_seq, 2 * EMBED_W), jnp.float32),
        scratch_types=[
            pltpu.VMEM((n_seq, LANES), jnp.int32),
            pltpu.VMEM((NBUF, LANES, 2 * EMBED_W), jnp.float32),
            pltpu.VMEM((NBUF, LANES, 1, EMBED_W), jnp.float32),
            pltpu.SemaphoreType.DMA,
            pltpu.SemaphoreType.DMA,
            pltpu.SemaphoreType.DMA,
            pltpu.SemaphoreType.DMA,
        ],
        compiler_params=pltpu.CompilerParams(
            use_tc_tiling_on_sc=False, needs_layout_passes=False
        ),
    )
    def lookup(lutp_hbm, xt_hbm, out_hbm, idx_v, rows_v, wb_v,
               g0, g1, w0, w1):
        wid = lax.axis_index("s") * nc + lax.axis_index("c")
        g_sems = [g0, g1]
        w_sems = [w0, w1]
        b0 = wid * LANES

        pltpu.sync_copy(xt_hbm.at[:, pl.ds(b0, LANES)], idx_v)

        for b in range(NBUF):
            pltpu.async_copy(
                lutp_hbm.at[idx_v.at[b]], rows_v.at[b], g_sems[b]
            )

        def outer(t, carry):
            for b in range(NBUF):
                s = t * NBUF + b
                pltpu.make_async_copy(
                    lutp_hbm.at[pl.ds(0, LANES)], rows_v.at[b], g_sems[b]
                ).wait()

                @pl.when(t > 0)
                def _drain(b=b):
                    pltpu.make_async_copy(
                        wb_v.at[b],
                        out_hbm.at[pl.ds(0, LANES), pl.ds(0, 1),
                                   pl.ds(0, EMBED_W)],
                        w_sems[b],
                    ).wait()

                def scale_grp(g, inner, b=b):
                    j0 = g * 16
                    for jj in range(16):
                        for q in range(EMBED_W // 16):
                            sl = pl.ds(q * 16, 16)
                            wb_v[b, j0 + jj, 0, sl] = (
                                rows_v[b, j0 + jj, sl] * SCALE
                            )
                    return inner

                lax.fori_loop(0, LANES // 16, scale_grp, 0)

                @pl.when(t + 1 < n_outer)
                def _refill(b=b, s=s):
                    pltpu.async_copy(
                        lutp_hbm.at[idx_v.at[s + NBUF]],
                        rows_v.at[b],
                        g_sems[b],
                    )

                pltpu.async_copy(
                    wb_v.at[b],
                    out_hbm.at[pl.ds(b0, LANES), pl.ds(s, 1),
                               pl.ds(0, EMBED_W)],
                    w_sems[b],
                )
            return carry

        lax.fori_loop(0, n_outer, outer, 0)

        for b in range(NBUF):
            pltpu.make_async_copy(
                wb_v.at[b],
                out_hbm.at[pl.ds(0, LANES), pl.ds(0, 1), pl.ds(0, EMBED_W)],
                w_sems[b],
            ).wait()

    return lookup


def kernel(x, lut):
    n_batch, n_seq = x.shape
    n_vocab, embed = lut.shape
    xt = x.T.astype(jnp.int32)
    lutp = jnp.pad(lut, ((0, 0), (0, embed)))
    outp = _make_lookup(n_seq, n_batch, n_vocab)(lutp, xt)
    return outp[:, :, :embed]
